# baseline (speedup 1.0000x reference)
import functools

import numpy as np
import jax
import jax.numpy as jnp
from jax import lax
from jax.experimental import pallas as pl
from jax.experimental.pallas import tpu as pltpu
from jax.experimental.pallas import tpu_sc as plsc

_E, _NT, _NA, _NE, _D, _H, _A = 8, 64, 16, 22, 128, 512, 32
_T = _NT * _NA
_B = 64
_NB = _T // _B + _E
_SLOTS = _NB * _B
_OC = 128


def _expert_block(x, ew1, eb1, ew2, eb2, lw1, lb1, lw2, lb2, vw1, vb1, vw2r, vb2):
    def bdot(a, b):
        return jnp.dot(a.astype(jnp.bfloat16), b.astype(jnp.bfloat16),
                       preferred_element_type=jnp.float32)

    def b32(a):
        return a.astype(jnp.bfloat16).astype(jnp.float32)

    h = jnp.maximum(bdot(x, ew1) + eb1, 0.0)
    v = bdot(h, ew2) + eb2
    ge = x.shape[0] // _B
    v3 = v.reshape(_B, ge, _H)
    v3b = b32(v3)
    eidx = lax.broadcasted_iota(jnp.int32, (_B, ge, 1), 1)
    vs3b = lax.slice(v3b, (0, 0, 0), (_B, 1, _H))
    score3 = jnp.sum(vs3b * v3b, axis=-1, keepdims=True) / np.sqrt(_H)

    def conc(lo, hi):
        mask = jnp.logical_and(eidx >= lo, eidx < hi)
        m = jnp.max(jnp.where(mask, score3, -1e30), axis=1, keepdims=True)
        ex = jnp.where(mask, jnp.exp(score3 - m), 0.0)
        attn = ex / jnp.sum(ex, axis=1, keepdims=True)
        v_c = jnp.sum(b32(attn) * v3b, axis=1)
        v_m = jnp.max(jnp.where(mask, v3, -1e30), axis=1)
        return v_c, v_m

    fc, fm = conc(1, 11)
    hc, hm = conc(11, _NE)
    v_c = jnp.concatenate([fc, hc], axis=-1)
    v_m = jnp.concatenate([fm, hm], axis=-1)
    hl = jnp.maximum(bdot(v_c, lw1) + lb1, 0.0)
    logits = bdot(hl, lw2) + lb2
    hv = jnp.maximum(bdot(v_m, vw1) + vb1, 0.0)
    value = jnp.sum(b32(hv) * b32(vw2r), axis=-1, keepdims=True) + vb2
    mx = jnp.max(logits, axis=-1, keepdims=True)
    ids = lax.broadcasted_iota(jnp.int32, (_B, _A), 1)
    act = jnp.min(jnp.where(logits == mx, ids, _A), axis=-1, keepdims=True)
    logp = -jnp.log(jnp.sum(jnp.exp(logits - mx), axis=-1, keepdims=True))
    col = lax.broadcasted_iota(jnp.int32, (_B, _OC), 1)
    return jnp.where(col == 0, act.astype(jnp.float32),
                     jnp.where(col == 1, value,
                               jnp.where(col == 2, logp, 0.0)))


_GE = 24


def _tc_forward(obs, be, bv, tokf,
                ew1, eb1, ew2, eb2, lw1, lb1, lw2, lb2, vw1, vb1, vw2r, vb2):
    def body(be_r, bv_r, tokf_r, obs_r,
             ew1_r, eb1_r, ew2_r, eb2_r, lw1_r, lb1_r, lw2_r, lb2_r,
             vw1_r, vb1_r, vw2_r, vb2_r, out_r, xbuf, sem):
        j = pl.program_id(0)

        def issue(jj):
            @pl.when(bv_r[jj] > 0)
            def _():
                buf = jj % 2

                def one(i, _):
                    t = tokf_r[jj, i]
                    pltpu.make_async_copy(
                        obs_r.at[t // _NA, t % _NA],
                        xbuf.at[buf, pl.ds(i * _GE, _NE)], sem.at[buf]).start()
                    return 0
                lax.fori_loop(0, _B, one, 0)

        def drain(jj):
            @pl.when(bv_r[jj] > 0)
            def _():
                buf = jj % 2

                def one(i, _):
                    pltpu.make_async_copy(
                        obs_r.at[0, 0],
                        xbuf.at[buf, pl.ds(i * _GE, _NE)], sem.at[buf]).wait()
                    return 0
                lax.fori_loop(0, _B, one, 0)

        @pl.when(j == 0)
        def _():
            xbuf[...] = jnp.zeros_like(xbuf)
            issue(0)

        @pl.when(j + 1 < _NB)
        def _():
            issue(j + 1)

        drain(j)

        @pl.when(bv_r[j] > 0)
        def _():
            out_r[...] = _expert_block(
                xbuf[j % 2], ew1_r[0], eb1_r[0], ew2_r[0], eb2_r[0],
                lw1_r[0], lb1_r[0], lw2_r[0], lb2_r[0],
                vw1_r[0], vb1_r[0], vw2_r[0], vb2_r[0])

    def w_idx(j, be_r, bv_r, tokf_r):
        return (be_r[j], 0, 0)

    grid_spec = pltpu.PrefetchScalarGridSpec(
        num_scalar_prefetch=3,
        grid=(_NB,),
        in_specs=[
            pl.BlockSpec(memory_space=pl.ANY),
            pl.BlockSpec((1, _D, _H), w_idx),
            pl.BlockSpec((1, 1, _H), w_idx),
            pl.BlockSpec((1, _H, _H), w_idx),
            pl.BlockSpec((1, 1, _H), w_idx),
            pl.BlockSpec((1, 2 * _H, _H), w_idx),
            pl.BlockSpec((1, 1, _H), w_idx),
            pl.BlockSpec((1, _H, _A), w_idx),
            pl.BlockSpec((1, 1, _A), w_idx),
            pl.BlockSpec((1, 2 * _H, _H), w_idx),
            pl.BlockSpec((1, 1, _H), w_idx),
            pl.BlockSpec((1, 1, _H), w_idx),
            pl.BlockSpec((1, 1, 1), w_idx),
        ],
        out_specs=pl.BlockSpec((_B, _OC), lambda j, be_r, bv_r, tokf_r: (j, 0)),
        scratch_shapes=[
            pltpu.VMEM((2, _B * _GE, _D), jnp.float32),
            pltpu.SemaphoreType.DMA((2,)),
        ],
    )
    return pl.pallas_call(
        body,
        grid_spec=grid_spec,
        out_shape=jax.ShapeDtypeStruct((_SLOTS, _OC), jnp.float32),
        compiler_params=pltpu.CompilerParams(dimension_semantics=("arbitrary",)),
    )(be, bv, tokf, obs, ew1, eb1, ew2, eb2, lw1, lb1, lw2, lb2, vw1, vb1, vw2r, vb2)


def _sc_gather_rows(table, idx, chunk_rows):
    _, d_w = table.shape
    n = idx.shape[0]
    info = plsc.get_sparse_core_info()
    n_w = info.num_cores * info.num_subcores
    rpw = n // n_w
    c_rows = min(chunk_rows, rpw)
    nchunks = rpw // c_rows
    mesh = plsc.VectorSubcoreMesh(core_axis_name="c", subcore_axis_name="s")

    @functools.partial(
        pl.kernel, mesh=mesh,
        out_type=jax.ShapeDtypeStruct((n, d_w), jnp.float32),
        scratch_types=[
            pltpu.VMEM((2, c_rows), jnp.int32),
            pltpu.VMEM((2, c_rows, d_w), jnp.float32),
            pltpu.SemaphoreType.DMA,
            pltpu.SemaphoreType.DMA,
        ],
    )
    def k(tab_h, idx_h, out_h, idx_v, rows_v, sem0, sem1):
        wid = lax.axis_index("s") * info.num_cores + lax.axis_index("c")
        base = wid * rpw
        sems = (sem0, sem1)
        pltpu.sync_copy(idx_h.at[pl.ds(base, c_rows)], idx_v.at[0])
        h_prev = pltpu.async_copy(tab_h.at[idx_v.at[0]], rows_v.at[0], sems[0])
        for c in range(nchunks):
            b = c % 2
            h_cur = h_prev
            if c + 1 < nchunks:
                nb = (c + 1) % 2
                pltpu.sync_copy(idx_h.at[pl.ds(base + (c + 1) * c_rows, c_rows)],
                                idx_v.at[nb])
                h_prev = pltpu.async_copy(tab_h.at[idx_v.at[nb]], rows_v.at[nb],
                                          sems[nb])
            h_cur.wait()
            pltpu.sync_copy(rows_v.at[b], out_h.at[pl.ds(base + c * c_rows, c_rows)])

    return k(table, idx)


def kernel(obs, expert_ids, enc_w1, enc_b1, enc_w2, enc_b2,
           log_w1, log_b1, log_w2, log_b2, val_w1, val_b1, val_w2, val_b2):
    eid = expert_ids.reshape(_T).astype(jnp.int32)

    onehot = (eid[:, None] == jnp.arange(_E, dtype=jnp.int32)[None, :]).astype(jnp.int32)
    cum = jnp.cumsum(onehot, axis=0)
    counts = cum[-1]
    pos = jnp.take_along_axis(cum, eid[:, None], axis=1)[:, 0] - 1
    nb_e = (counts + _B - 1) // _B
    cnb = jnp.cumsum(nb_e)
    bstart = jnp.concatenate([jnp.zeros((1,), jnp.int32), cnb[:-1]])
    total = cnb[-1]
    slot_t = (bstart[eid] + pos // _B) * _B + (pos % _B)
    tok_of_slot = jnp.zeros((_SLOTS,), jnp.int32).at[slot_t].set(
        jnp.arange(_T, dtype=jnp.int32))
    jarr = jnp.arange(_NB, dtype=jnp.int32)
    ofi = jnp.minimum(jarr, total - 1)
    be = (jnp.sum((ofi[:, None] >= bstart[None, :]).astype(jnp.int32), axis=1) - 1)
    bv = jnp.where(jarr < total,
                   jnp.clip(counts[be] - (ofi - bstart[be]) * _B, 0, _B), 0)

    tokf = tok_of_slot.reshape(_NB, _B)
    eb1r = enc_b1.reshape(_E, 1, _H)
    eb2r = enc_b2.reshape(_E, 1, _H)
    lb1r = log_b1.reshape(_E, 1, _H)
    lb2r = log_b2.reshape(_E, 1, _A)
    vb1r = val_b1.reshape(_E, 1, _H)
    vb2r = val_b2.reshape(_E, 1, 1)
    vw2r = val_w2.reshape(_E, 1, _H)
    out_sorted = _tc_forward(obs, be, bv, tokf,
                             enc_w1, eb1r, enc_w2, eb2r,
                             log_w1, lb1r, log_w2, lb2r,
                             val_w1, vb1r, vw2r, vb2r)

    fin = _sc_gather_rows(out_sorted, slot_t, 32)
    act = fin[:, 0].astype(jnp.int32).reshape(_NT, _NA)
    value = fin[:, 1].reshape(_NT, _NA, 1)
    logp = fin[:, 2].reshape(_NT, _NA)
    return act, value, logp


# speedup vs baseline: 1.2624x; 1.2624x over previous
import functools

import numpy as np
import jax
import jax.numpy as jnp
from jax import lax
from jax.experimental import pallas as pl
from jax.experimental.pallas import tpu as pltpu
from jax.experimental.pallas import tpu_sc as plsc

_E, _NT, _NA, _NE, _D, _H, _A = 8, 64, 16, 22, 128, 512, 32
_T = _NT * _NA
_B = 64
_NB = _T // _B + _E
_SLOTS = _NB * _B
_OC = 128


def _expert_block(x, ew1, eb1, ew2, eb2, lw1, lb1, lw2, lb2, vw1, vb1, vw2r, vb2):
    def bdot(a, b):
        return jnp.dot(a.astype(jnp.bfloat16), b.astype(jnp.bfloat16),
                       preferred_element_type=jnp.float32)

    def b32(a):
        return a.astype(jnp.bfloat16).astype(jnp.float32)

    h = jnp.maximum(bdot(x, ew1) + eb1, 0.0)
    v = bdot(h, ew2) + eb2
    ge = x.shape[0] // _B
    v3 = v.reshape(_B, ge, _H)
    v3b = b32(v3)
    eidx = lax.broadcasted_iota(jnp.int32, (_B, ge, 1), 1)
    vs3b = lax.slice(v3b, (0, 0, 0), (_B, 1, _H))
    score3 = jnp.sum(vs3b * v3b, axis=-1, keepdims=True) / np.sqrt(_H)

    def conc(lo, hi):
        mask = jnp.logical_and(eidx >= lo, eidx < hi)
        m = jnp.max(jnp.where(mask, score3, -1e30), axis=1, keepdims=True)
        ex = jnp.where(mask, jnp.exp(score3 - m), 0.0)
        attn = ex / jnp.sum(ex, axis=1, keepdims=True)
        v_c = jnp.sum(b32(attn) * v3b, axis=1)
        v_m = jnp.max(jnp.where(mask, v3, -1e30), axis=1)
        return v_c, v_m

    fc, fm = conc(1, 11)
    hc, hm = conc(11, _NE)
    v_c = jnp.concatenate([fc, hc], axis=-1)
    v_m = jnp.concatenate([fm, hm], axis=-1)
    hl = jnp.maximum(bdot(v_c, lw1) + lb1, 0.0)
    logits = bdot(hl, lw2) + lb2
    hv = jnp.maximum(bdot(v_m, vw1) + vb1, 0.0)
    value = jnp.sum(b32(hv) * b32(vw2r), axis=-1, keepdims=True) + vb2
    mx = jnp.max(logits, axis=-1, keepdims=True)
    ids = lax.broadcasted_iota(jnp.int32, (_B, _A), 1)
    act = jnp.min(jnp.where(logits == mx, ids, _A), axis=-1, keepdims=True)
    logp = -jnp.log(jnp.sum(jnp.exp(logits - mx), axis=-1, keepdims=True))
    col = lax.broadcasted_iota(jnp.int32, (_B, _OC), 1)
    return jnp.where(col == 0, act.astype(jnp.float32),
                     jnp.where(col == 1, value,
                               jnp.where(col == 2, logp, 0.0)))


_GE = 24


def _tc_forward(obs, be, bv, tokf,
                ew1, eb1, ew2, eb2, lw1, lb1, lw2, lb2, vw1, vb1, vw2r, vb2):
    def body(be_r, bv_r, tokf_r, obs_r,
             ew1_r, eb1_r, ew2_r, eb2_r, lw1_r, lb1_r, lw2_r, lb2_r,
             vw1_r, vb1_r, vw2_r, vb2_r, out_r, xbuf, sem):
        j = pl.program_id(0)

        def issue(jj):
            @pl.when(bv_r[jj] > 0)
            def _():
                buf = jj % 2
                for i in range(_B):
                    pltpu.make_async_copy(
                        obs_r.at[tokf_r[jj, i]],
                        xbuf.at[buf, pl.ds(i * _GE, _NE)], sem.at[buf]).start()

        def drain(jj):
            @pl.when(bv_r[jj] > 0)
            def _():
                buf = jj % 2
                pltpu.make_async_copy(
                    xbuf.at[(buf + 1) % 2, pl.ds(0, _B * _NE)],
                    xbuf.at[buf, pl.ds(0, _B * _NE)], sem.at[buf]).wait()

        @pl.when(j == 0)
        def _():
            xbuf[...] = jnp.zeros_like(xbuf)
            issue(0)

        @pl.when(j + 1 < _NB)
        def _():
            issue(j + 1)

        drain(j)

        @pl.when(bv_r[j] > 0)
        def _():
            out_r[...] = _expert_block(
                xbuf[j % 2], ew1_r[0], eb1_r[0], ew2_r[0], eb2_r[0],
                lw1_r[0], lb1_r[0], lw2_r[0], lb2_r[0],
                vw1_r[0], vb1_r[0], vw2_r[0], vb2_r[0])

    def w_idx(j, be_r, bv_r, tokf_r):
        return (be_r[j], 0, 0)

    grid_spec = pltpu.PrefetchScalarGridSpec(
        num_scalar_prefetch=3,
        grid=(_NB,),
        in_specs=[
            pl.BlockSpec(memory_space=pl.ANY),
            pl.BlockSpec((1, _D, _H), w_idx),
            pl.BlockSpec((1, 1, _H), w_idx),
            pl.BlockSpec((1, _H, _H), w_idx),
            pl.BlockSpec((1, 1, _H), w_idx),
            pl.BlockSpec((1, 2 * _H, _H), w_idx),
            pl.BlockSpec((1, 1, _H), w_idx),
            pl.BlockSpec((1, _H, _A), w_idx),
            pl.BlockSpec((1, 1, _A), w_idx),
            pl.BlockSpec((1, 2 * _H, _H), w_idx),
            pl.BlockSpec((1, 1, _H), w_idx),
            pl.BlockSpec((1, 1, _H), w_idx),
            pl.BlockSpec((1, 1, 1), w_idx),
        ],
        out_specs=pl.BlockSpec((_B, _OC), lambda j, be_r, bv_r, tokf_r: (j, 0)),
        scratch_shapes=[
            pltpu.VMEM((2, _B * _GE, _D), jnp.float32),
            pltpu.SemaphoreType.DMA((2,)),
        ],
    )
    return pl.pallas_call(
        body,
        grid_spec=grid_spec,
        out_shape=jax.ShapeDtypeStruct((_SLOTS, _OC), jnp.float32),
        compiler_params=pltpu.CompilerParams(dimension_semantics=("arbitrary",)),
    )(be, bv, tokf, obs, ew1, eb1, ew2, eb2, lw1, lb1, lw2, lb2, vw1, vb1, vw2r, vb2)


def _sc_gather_rows(table, idx, chunk_rows):
    _, d_w = table.shape
    n = idx.shape[0]
    info = plsc.get_sparse_core_info()
    n_w = info.num_cores * info.num_subcores
    rpw = n // n_w
    c_rows = min(chunk_rows, rpw)
    nchunks = rpw // c_rows
    mesh = plsc.VectorSubcoreMesh(core_axis_name="c", subcore_axis_name="s")

    @functools.partial(
        pl.kernel, mesh=mesh,
        out_type=jax.ShapeDtypeStruct((n, d_w), jnp.float32),
        scratch_types=[
            pltpu.VMEM((2, c_rows), jnp.int32),
            pltpu.VMEM((2, c_rows, d_w), jnp.float32),
            pltpu.SemaphoreType.DMA,
            pltpu.SemaphoreType.DMA,
        ],
    )
    def k(tab_h, idx_h, out_h, idx_v, rows_v, sem0, sem1):
        wid = lax.axis_index("s") * info.num_cores + lax.axis_index("c")
        base = wid * rpw
        sems = (sem0, sem1)
        pltpu.sync_copy(idx_h.at[pl.ds(base, c_rows)], idx_v.at[0])
        h_prev = pltpu.async_copy(tab_h.at[idx_v.at[0]], rows_v.at[0], sems[0])
        for c in range(nchunks):
            b = c % 2
            h_cur = h_prev
            if c + 1 < nchunks:
                nb = (c + 1) % 2
                pltpu.sync_copy(idx_h.at[pl.ds(base + (c + 1) * c_rows, c_rows)],
                                idx_v.at[nb])
                h_prev = pltpu.async_copy(tab_h.at[idx_v.at[nb]], rows_v.at[nb],
                                          sems[nb])
            h_cur.wait()
            pltpu.sync_copy(rows_v.at[b], out_h.at[pl.ds(base + c * c_rows, c_rows)])

    return k(table, idx)


def kernel(obs, expert_ids, enc_w1, enc_b1, enc_w2, enc_b2,
           log_w1, log_b1, log_w2, log_b2, val_w1, val_b1, val_w2, val_b2):
    eid = expert_ids.reshape(_T).astype(jnp.int32)

    onehot = (eid[:, None] == jnp.arange(_E, dtype=jnp.int32)[None, :]).astype(jnp.int32)
    cum = jnp.cumsum(onehot, axis=0)
    counts = cum[-1]
    pos = jnp.take_along_axis(cum, eid[:, None], axis=1)[:, 0] - 1
    nb_e = (counts + _B - 1) // _B
    cnb = jnp.cumsum(nb_e)
    bstart = jnp.concatenate([jnp.zeros((1,), jnp.int32), cnb[:-1]])
    total = cnb[-1]
    slot_t = (bstart[eid] + pos // _B) * _B + (pos % _B)
    tok_of_slot = jnp.zeros((_SLOTS,), jnp.int32).at[slot_t].set(
        jnp.arange(_T, dtype=jnp.int32))
    jarr = jnp.arange(_NB, dtype=jnp.int32)
    ofi = jnp.minimum(jarr, total - 1)
    be = (jnp.sum((ofi[:, None] >= bstart[None, :]).astype(jnp.int32), axis=1) - 1)
    bv = jnp.where(jarr < total,
                   jnp.clip(counts[be] - (ofi - bstart[be]) * _B, 0, _B), 0)

    tokf = tok_of_slot.reshape(_NB, _B)
    eb1r = enc_b1.reshape(_E, 1, _H)
    eb2r = enc_b2.reshape(_E, 1, _H)
    lb1r = log_b1.reshape(_E, 1, _H)
    lb2r = log_b2.reshape(_E, 1, _A)
    vb1r = val_b1.reshape(_E, 1, _H)
    vb2r = val_b2.reshape(_E, 1, 1)
    vw2r = val_w2.reshape(_E, 1, _H)
    out_sorted = _tc_forward(obs.reshape(_T, _NE, _D), be, bv, tokf,
                             enc_w1, eb1r, enc_w2, eb2r,
                             log_w1, lb1r, log_w2, lb2r,
                             val_w1, vb1r, vw2r, vb2r)

    fin = _sc_gather_rows(out_sorted, slot_t, 32)
    act = fin[:, 0].astype(jnp.int32).reshape(_NT, _NA)
    value = fin[:, 1].reshape(_NT, _NA, 1)
    logp = fin[:, 2].reshape(_NT, _NA)
    return act, value, logp


# speedup vs baseline: 1.4132x; 1.1194x over previous
import functools

import numpy as np
import jax
import jax.numpy as jnp
from jax import lax
from jax.experimental import pallas as pl
from jax.experimental.pallas import tpu as pltpu
from jax.experimental.pallas import tpu_sc as plsc

_E, _NT, _NA, _NE, _D, _H, _A = 8, 64, 16, 22, 128, 512, 32
_T = _NT * _NA
_B = 64
_NB = _T // _B + _E
_SLOTS = _NB * _B
_OC = 128


def _expert_block(x, ew1, eb1, ew2, eb2, lw1, lb1, lw2, lb2, vw1, vb1, vw2r, vb2):
    def bdot(a, b):
        return jnp.dot(a.astype(jnp.bfloat16), b.astype(jnp.bfloat16),
                       preferred_element_type=jnp.float32)

    def b32(a):
        return a.astype(jnp.bfloat16).astype(jnp.float32)

    h = jnp.maximum(bdot(x, ew1) + eb1, 0.0)
    v = bdot(h, ew2) + eb2
    ge = x.shape[0] // _B
    v3 = v.reshape(_B, ge, _H)
    v3b = b32(v3)
    eidx = lax.broadcasted_iota(jnp.int32, (_B, ge, 1), 1)
    vs3b = lax.slice(v3b, (0, 0, 0), (_B, 1, _H))
    score3 = jnp.sum(vs3b * v3b, axis=-1, keepdims=True) / np.sqrt(_H)

    def conc(lo, hi):
        mask = jnp.logical_and(eidx >= lo, eidx < hi)
        m = jnp.max(jnp.where(mask, score3, -1e30), axis=1, keepdims=True)
        ex = jnp.where(mask, jnp.exp(score3 - m), 0.0)
        attn = ex / jnp.sum(ex, axis=1, keepdims=True)
        v_c = jnp.sum(b32(attn) * v3b, axis=1)
        v_m = jnp.max(jnp.where(mask, v3, -1e30), axis=1)
        return v_c, v_m

    fc, fm = conc(1, 11)
    hc, hm = conc(11, _NE)
    v_c = jnp.concatenate([fc, hc], axis=-1)
    v_m = jnp.concatenate([fm, hm], axis=-1)
    hl = jnp.maximum(bdot(v_c, lw1) + lb1, 0.0)
    logits = bdot(hl, lw2) + lb2
    hv = jnp.maximum(bdot(v_m, vw1) + vb1, 0.0)
    value = jnp.sum(b32(hv) * b32(vw2r), axis=-1, keepdims=True) + vb2
    mx = jnp.max(logits, axis=-1, keepdims=True)
    ids = lax.broadcasted_iota(jnp.int32, (_B, _A), 1)
    act = jnp.min(jnp.where(logits == mx, ids, _A), axis=-1, keepdims=True)
    logp = -jnp.log(jnp.sum(jnp.exp(logits - mx), axis=-1, keepdims=True))
    col = lax.broadcasted_iota(jnp.int32, (_B, _OC), 1)
    return jnp.where(col == 0, act.astype(jnp.float32),
                     jnp.where(col == 1, value,
                               jnp.where(col == 2, logp, 0.0)))


_GE = 24


def _tc_forward(obs, be, bv, tokf,
                ew1, eb1, ew2, eb2, lw1, lb1, lw2, lb2, vw1, vb1, vw2r, vb2):
    def body(be_r, bv_r, tokf_r, obs_r,
             ew1_r, eb1_r, ew2_r, eb2_r, lw1_r, lb1_r, lw2_r, lb2_r,
             vw1_r, vb1_r, vw2_r, vb2_r, out_r, xbuf, sem):
        j = pl.program_id(0)

        def issue(jj):
            @pl.when(bv_r[0, jj] > 0)
            def _():
                buf = jj % 2
                for i in range(_B):
                    pltpu.make_async_copy(
                        obs_r.at[tokf_r[0, jj * _B + i]],
                        xbuf.at[buf, pl.ds(i * _GE, _NE)], sem.at[buf]).start()

        def drain(jj):
            @pl.when(bv_r[0, jj] > 0)
            def _():
                buf = jj % 2
                pltpu.make_async_copy(
                    xbuf.at[(buf + 1) % 2, pl.ds(0, _B * _NE)],
                    xbuf.at[buf, pl.ds(0, _B * _NE)], sem.at[buf]).wait()

        @pl.when(j == 0)
        def _():
            xbuf[...] = jnp.zeros_like(xbuf)
            issue(0)

        @pl.when(j + 1 < _NB)
        def _():
            issue(j + 1)

        drain(j)

        @pl.when(bv_r[0, j] > 0)
        def _():
            out_r[...] = _expert_block(
                xbuf[j % 2], ew1_r[0], eb1_r[0], ew2_r[0], eb2_r[0],
                lw1_r[0], lb1_r[0], lw2_r[0], lb2_r[0],
                vw1_r[0], vb1_r[0], vw2_r[0], vb2_r[0])

    def w_idx(j, be_r, bv_r, tokf_r):
        return (be_r[0, j], 0, 0)

    grid_spec = pltpu.PrefetchScalarGridSpec(
        num_scalar_prefetch=3,
        grid=(_NB,),
        in_specs=[
            pl.BlockSpec(memory_space=pl.ANY),
            pl.BlockSpec((1, _D, _H), w_idx),
            pl.BlockSpec((1, 1, _H), w_idx),
            pl.BlockSpec((1, _H, _H), w_idx),
            pl.BlockSpec((1, 1, _H), w_idx),
            pl.BlockSpec((1, 2 * _H, _H), w_idx),
            pl.BlockSpec((1, 1, _H), w_idx),
            pl.BlockSpec((1, _H, _A), w_idx),
            pl.BlockSpec((1, 1, _A), w_idx),
            pl.BlockSpec((1, 2 * _H, _H), w_idx),
            pl.BlockSpec((1, 1, _H), w_idx),
            pl.BlockSpec((1, 1, _H), w_idx),
            pl.BlockSpec((1, 1, 1), w_idx),
        ],
        out_specs=pl.BlockSpec((_B, _OC), lambda j, be_r, bv_r, tokf_r: (j, 0)),
        scratch_shapes=[
            pltpu.VMEM((2, _B * _GE, _D), jnp.float32),
            pltpu.SemaphoreType.DMA((2,)),
        ],
    )
    return pl.pallas_call(
        body,
        grid_spec=grid_spec,
        out_shape=jax.ShapeDtypeStruct((_SLOTS, _OC), jnp.float32),
        compiler_params=pltpu.CompilerParams(dimension_semantics=("arbitrary",)),
    )(be, bv, tokf, obs, ew1, eb1, ew2, eb2, lw1, lb1, lw2, lb2, vw1, vb1, vw2r, vb2)


def _route_meta(eid_col):
    def body(eid_r, be_r, bv_r, tok_r, slot_r):
        eidc = eid_r[...]
        onehot = (eidc == lax.broadcasted_iota(jnp.int32, (_T, _E), 1)
                  ).astype(jnp.int32)
        cum = onehot
        s = 1
        while s < _T:
            shifted = lax.pad(cum, jnp.int32(0),
                              ((s, 0, 0), (0, 0, 0)))[:_T]
            cum = cum + shifted
            s *= 2
        counts = [lax.slice(cum, (_T - 1, e), (_T, e + 1)) for e in range(_E)]
        nb = [(c + (_B - 1)) // _B for c in counts]
        bst = [jnp.zeros((1, 1), jnp.int32)]
        for e in range(_E):
            bst.append(bst[e] + nb[e])
        total = bst[_E]
        pos = jnp.sum(cum * onehot, axis=1, keepdims=True) - 1
        bstart_tok = jnp.zeros((_T, 1), jnp.int32)
        for e in range(_E):
            bstart_tok = bstart_tok + lax.slice(onehot, (0, e), (_T, e + 1)) * bst[e]
        slot_col = (bstart_tok + pos // _B) * _B + (pos % _B)
        slot_r[...] = slot_col.reshape(1, _T)
        jlane = lax.broadcasted_iota(jnp.int32, (1, _NB), 1)
        be = jnp.zeros((1, _NB), jnp.int32)
        for e in range(_E):
            be = be + (jlane >= bst[e]).astype(jnp.int32)
        be = jnp.clip(be - 1, 0, _E - 1)
        cnt_at = jnp.zeros((1, _NB), jnp.int32)
        bst_at = jnp.zeros((1, _NB), jnp.int32)
        for e in range(_E):
            sel = (be == e).astype(jnp.int32)
            cnt_at = cnt_at + sel * counts[e]
            bst_at = bst_at + sel * bst[e]
        bv = jnp.where(jlane < total,
                       jnp.clip(cnt_at - (jlane - bst_at) * _B, 0, _B), 0)
        be_r[...] = be
        bv_r[...] = bv
        pmat = jnp.where(slot_col == lax.broadcasted_iota(jnp.int32, (_T, _SLOTS), 1),
                         1.0, 0.0)
        ti = lax.broadcasted_iota(jnp.int32, (1, _T), 1).astype(jnp.float32)
        tok_f = jnp.dot(ti, pmat, preferred_element_type=jnp.float32,
                        precision=lax.Precision.HIGHEST)
        tok_r[...] = tok_f.astype(jnp.int32)

    return pl.pallas_call(
        body,
        grid=(1,),
        out_shape=(
            jax.ShapeDtypeStruct((1, _NB), jnp.int32),
            jax.ShapeDtypeStruct((1, _NB), jnp.int32),
            jax.ShapeDtypeStruct((1, _SLOTS), jnp.int32),
            jax.ShapeDtypeStruct((1, _T), jnp.int32),
        ),
    )(eid_col)


def _sc_gather_rows(table, idx, chunk_rows):
    _, d_w = table.shape
    n = idx.shape[1]
    info = plsc.get_sparse_core_info()
    n_w = info.num_cores * info.num_subcores
    rpw = n // n_w
    c_rows = min(chunk_rows, rpw)
    nchunks = rpw // c_rows
    mesh = plsc.VectorSubcoreMesh(core_axis_name="c", subcore_axis_name="s")

    @functools.partial(
        pl.kernel, mesh=mesh,
        out_type=jax.ShapeDtypeStruct((n, d_w), jnp.float32),
        scratch_types=[
            pltpu.VMEM((2, c_rows), jnp.int32),
            pltpu.VMEM((2, c_rows, d_w), jnp.float32),
            pltpu.SemaphoreType.DMA,
            pltpu.SemaphoreType.DMA,
        ],
    )
    def k(tab_h, idx_h, out_h, idx_v, rows_v, sem0, sem1):
        wid = lax.axis_index("s") * info.num_cores + lax.axis_index("c")
        base = wid * rpw
        sems = (sem0, sem1)
        pltpu.sync_copy(idx_h.at[0, pl.ds(base, c_rows)], idx_v.at[0])
        h_prev = pltpu.async_copy(tab_h.at[idx_v.at[0]], rows_v.at[0], sems[0])
        for c in range(nchunks):
            b = c % 2
            h_cur = h_prev
            if c + 1 < nchunks:
                nb = (c + 1) % 2
                pltpu.sync_copy(idx_h.at[0, pl.ds(base + (c + 1) * c_rows, c_rows)],
                                idx_v.at[nb])
                h_prev = pltpu.async_copy(tab_h.at[idx_v.at[nb]], rows_v.at[nb],
                                          sems[nb])
            h_cur.wait()
            pltpu.sync_copy(rows_v.at[b], out_h.at[pl.ds(base + c * c_rows, c_rows)])

    return k(table, idx)


def kernel(obs, expert_ids, enc_w1, enc_b1, enc_w2, enc_b2,
           log_w1, log_b1, log_w2, log_b2, val_w1, val_b1, val_w2, val_b2):
    eid_col = expert_ids.reshape(_T, 1).astype(jnp.int32)

    be, bv, tokf, slot_row = _route_meta(eid_col)

    eb1r = enc_b1.reshape(_E, 1, _H)
    eb2r = enc_b2.reshape(_E, 1, _H)
    lb1r = log_b1.reshape(_E, 1, _H)
    lb2r = log_b2.reshape(_E, 1, _A)
    vb1r = val_b1.reshape(_E, 1, _H)
    vb2r = val_b2.reshape(_E, 1, 1)
    vw2r = val_w2.reshape(_E, 1, _H)
    out_sorted = _tc_forward(obs.reshape(_T, _NE, _D), be, bv, tokf,
                             enc_w1, eb1r, enc_w2, eb2r,
                             log_w1, lb1r, log_w2, lb2r,
                             val_w1, vb1r, vw2r, vb2r)

    fin = _sc_gather_rows(out_sorted, slot_row, 32)
    act = fin[:, 0].astype(jnp.int32).reshape(_NT, _NA)
    value = fin[:, 1].reshape(_NT, _NA, 1)
    logp = fin[:, 2].reshape(_NT, _NA)
    return act, value, logp
